# baseline (speedup 1.0000x reference)
import jax
import jax.numpy as jnp
from jax.experimental import pallas as pl
from jax.experimental.pallas import tpu as pltpu

T = 2048
D = 1024
H = 4096
S = 8
K = 2
A = T * K
BR = 256
NB = A // BR + S - 1
NP = NB * BR
EPS = 1e-5

BH = 1024
NH = H // BH
BT2 = 256
NT2 = T // BT2

_DOT = jnp.bfloat16


def _layer_norm(x, g, b):
    mu = jnp.mean(x, axis=-1, keepdims=True)
    var = jnp.mean((x - mu) ** 2, axis=-1, keepdims=True)
    return (x - mu) * jax.lax.rsqrt(var + EPS) * g + b


def _gelu(x):
    return 0.5 * x * (1.0 + jax.lax.erf(x * 0.7071067811865476))


def _router_kernel(x_ref, g1_ref, b1_ref, wr_ref, br_ref, normed_ref,
                   i12_ref, w12_ref):
    x = x_ref[...]
    normed = _layer_norm(x, g1_ref[...], b1_ref[...])
    normed_ref[...] = normed.astype(normed_ref.dtype)
    logits = jnp.dot(normed, wr_ref[...], preferred_element_type=jnp.float32)
    logits = logits + br_ref[...]
    iota = jax.lax.broadcasted_iota(jnp.int32, logits.shape, 1)
    v1 = jnp.max(logits, axis=-1, keepdims=True)
    i1 = jnp.min(jnp.where(logits == v1, iota, S), axis=-1, keepdims=True)
    l2 = jnp.where(iota == i1, -jnp.inf, logits)
    v2 = jnp.max(l2, axis=-1, keepdims=True)
    i2 = jnp.min(jnp.where(l2 == v2, iota, S), axis=-1, keepdims=True)
    e2 = jnp.exp(v2 - v1)
    w1 = 1.0 / (1.0 + e2)
    w2 = e2 * w1
    i12_ref[...] = jnp.concatenate([i1, i2], axis=1)
    w12_ref[...] = jnp.concatenate([w1, w2], axis=1)


def _group_kernel(be_ref, normed_ref, src_ref, w1_ref, b1_ref, w2_ref,
                  b2_ref, out_ref, xscr_ref):
    hb = pl.program_id(0)
    j = pl.program_id(1)
    row = j * BR

    @pl.when(hb == 0)
    def _dispatch():
        src = src_ref[:, 0:1]
        t_iota = jax.lax.broadcasted_iota(jnp.int32, (BR, T), 1)
        onehot = jnp.where(src == t_iota, 1.0, 0.0).astype(_DOT)
        rows = jnp.dot(onehot, normed_ref[...],
                       preferred_element_type=jnp.float32)
        xscr_ref[pl.ds(row, BR), :] = rows.astype(_DOT)

    xb = xscr_ref[pl.ds(row, BR), :]
    h1 = jnp.dot(xb, w1_ref[0].astype(_DOT),
                 preferred_element_type=jnp.float32)
    h1 = h1 + b1_ref[0]
    g = _gelu(h1).astype(_DOT)
    y = jnp.dot(g, w2_ref[0].astype(_DOT), preferred_element_type=jnp.float32)

    @pl.when(hb == 0)
    def _init():
        out_ref[pl.ds(row, BR), :] = (y + b2_ref[0]).astype(out_ref.dtype)

    @pl.when(hb != 0)
    def _acc():
        out_ref[pl.ds(row, BR), :] += y.astype(out_ref.dtype)


def _dense_kernel(x_ref, ys_ref, src_ref, ws_ref, g2_ref, b2_ref, wd1_ref,
                  bd1_ref, wd2_ref, bd2_ref, wg_ref, bg_ref, out_ref):
    t = pl.program_id(0)
    x = x_ref[...]
    src_row = src_ref[0]
    w_row = ws_ref[0]
    t_iota = jax.lax.broadcasted_iota(jnp.int32, (BT2, NP), 0) + t * BT2
    mask = jnp.where(src_row == t_iota, w_row, 0.0).astype(_DOT)
    so = jnp.dot(mask, ys_ref[...], preferred_element_type=jnp.float32)
    x1 = x + so
    x1n = _layer_norm(x1, g2_ref[...], b2_ref[...])
    gate_logit = jnp.sum(x1n * wg_ref[...], axis=-1, keepdims=True) + bg_ref[0, 0]
    gate = jax.nn.sigmoid(gate_logit)
    h = jnp.dot(x1n.astype(_DOT), wd1_ref[...].astype(_DOT),
                preferred_element_type=jnp.float32) + bd1_ref[...]
    g = _gelu(h).astype(_DOT)
    do = jnp.dot(g, wd2_ref[...].astype(_DOT),
                 preferred_element_type=jnp.float32)
    do = do + bd2_ref[...]
    out_ref[0] = x1 + gate * so + (1.0 - gate) * do


def _schedule(i12, w12):
    e = i12.reshape(A)
    oh = (e[:, None] == jnp.arange(S)[None, :]).astype(jnp.int32)
    csum = jnp.cumsum(oh, axis=0)
    rank = jnp.sum(oh * csum, axis=1) - 1
    counts = csum[-1]
    nblk = (counts + BR - 1) // BR
    end_blk = jnp.cumsum(nblk)
    starts = (end_blk - nblk) * BR
    pos = starts[e] + rank
    src_pad = jnp.full((NP,), -1, jnp.int32).at[pos].set(
        jnp.arange(A, dtype=jnp.int32) // K)
    w_sorted = jnp.zeros((NP,), jnp.float32).at[pos].set(w12.reshape(A))
    block_expert = jnp.minimum(
        jnp.searchsorted(end_blk, jnp.arange(NB, dtype=jnp.int32),
                         side="right"),
        S - 1).astype(jnp.int32)
    return src_pad, w_sorted, block_expert


def kernel(x, gamma1, beta1, gamma2, beta2, Wr, br, W1e, b1e, W2e, b2e,
           Wd1, bd1, Wd2, bd2, Wg, bg):
    x2d = x.reshape(T, D)

    normed, i12, w12 = pl.pallas_call(
        _router_kernel,
        out_shape=(
            jax.ShapeDtypeStruct((T, D), _DOT),
            jax.ShapeDtypeStruct((T, K), jnp.int32),
            jax.ShapeDtypeStruct((T, K), jnp.float32),
        ),
    )(x2d, gamma1.reshape(1, D), beta1.reshape(1, D), Wr, br.reshape(1, S))

    src_pad, w_sorted, block_expert = _schedule(i12, w12)
    src8 = jnp.broadcast_to(src_pad[:, None], (NP, 8))

    y_sorted = pl.pallas_call(
        _group_kernel,
        grid_spec=pltpu.PrefetchScalarGridSpec(
            num_scalar_prefetch=1,
            grid=(NH, NB),
            in_specs=[
                pl.BlockSpec((T, D), lambda hb, j, be: (0, 0)),
                pl.BlockSpec((BR, 8), lambda hb, j, be: (j, 0)),
                pl.BlockSpec((1, D, BH), lambda hb, j, be: (be[j], 0, hb)),
                pl.BlockSpec((1, 1, BH), lambda hb, j, be: (be[j], 0, hb)),
                pl.BlockSpec((1, BH, D), lambda hb, j, be: (be[j], hb, 0)),
                pl.BlockSpec((1, 1, D), lambda hb, j, be: (be[j], 0, 0)),
            ],
            out_specs=pl.BlockSpec((NP, D), lambda hb, j, be: (0, 0)),
            scratch_shapes=[pltpu.VMEM((NP, D), _DOT)],
        ),
        out_shape=jax.ShapeDtypeStruct((NP, D), _DOT),
        compiler_params=pltpu.CompilerParams(
            dimension_semantics=("arbitrary", "arbitrary"),
        ),
    )(block_expert, normed, src8, W1e, b1e.reshape(S, 1, H), W2e,
      b2e.reshape(S, 1, D))

    out = pl.pallas_call(
        _dense_kernel,
        grid=(NT2,),
        in_specs=[
            pl.BlockSpec((BT2, D), lambda t: (t, 0)),
            pl.BlockSpec((NP, D), lambda t: (0, 0)),
            pl.BlockSpec((1, 1, NP), lambda t: (0, 0, 0)),
            pl.BlockSpec((1, 1, NP), lambda t: (0, 0, 0)),
            pl.BlockSpec((1, D), lambda t: (0, 0)),
            pl.BlockSpec((1, D), lambda t: (0, 0)),
            pl.BlockSpec((D, H), lambda t: (0, 0)),
            pl.BlockSpec((1, H), lambda t: (0, 0)),
            pl.BlockSpec((H, D), lambda t: (0, 0)),
            pl.BlockSpec((1, D), lambda t: (0, 0)),
            pl.BlockSpec((1, D), lambda t: (0, 0)),
            pl.BlockSpec((1, 1), lambda t: (0, 0)),
        ],
        out_specs=pl.BlockSpec((1, BT2, D), lambda t: (0, t, 0)),
        out_shape=jax.ShapeDtypeStruct((1, T, D), jnp.float32),
        compiler_params=pltpu.CompilerParams(
            dimension_semantics=("arbitrary",),
        ),
    )(
        x2d,
        y_sorted,
        src_pad.reshape(1, 1, NP),
        w_sorted.reshape(1, 1, NP),
        gamma2.reshape(1, D),
        beta2.reshape(1, D),
        Wd1,
        bd1.reshape(1, H),
        Wd2,
        bd2.reshape(1, D),
        Wg.reshape(1, D),
        bg.reshape(1, 1),
    )
    return out


# speedup vs baseline: 1.1537x; 1.1537x over previous
import jax
import jax.numpy as jnp
from jax.experimental import pallas as pl
from jax.experimental.pallas import tpu as pltpu

T = 2048
D = 1024
H = 4096
S = 8
K = 2
A = T * K
BR = 256
NB = A // BR + S - 1
NP = NB * BR
NBP = 32
CH = 512
EPS = 1e-5

BH = 1024
NH = H // BH
BT2 = 256
NT2 = T // BT2

_DOT = jnp.bfloat16


def _layer_norm(x, g, b):
    mu = jnp.mean(x, axis=-1, keepdims=True)
    var = jnp.mean((x - mu) ** 2, axis=-1, keepdims=True)
    return (x - mu) * jax.lax.rsqrt(var + EPS) * g + b


def _gelu(x):
    return 0.5 * x * (1.0 + jax.lax.erf(x * 0.7071067811865476))


def _lane_cumsum8(v):
    for sh in (1, 2, 4):
        v = v + jnp.concatenate(
            [jnp.zeros((1, sh), v.dtype), v[:, : S - sh]], axis=1)
    return v


def _router_kernel(x_ref, g1_ref, b1_ref, wr_ref, br_ref,
                   normed_ref, pos_ref, w12_ref, be_ref):
    x = x_ref[...]
    normed = _layer_norm(x, g1_ref[...], b1_ref[...])
    normed_ref[...] = normed.astype(normed_ref.dtype)
    logits = jnp.dot(normed, wr_ref[...], preferred_element_type=jnp.float32)
    logits = logits + br_ref[...]
    iota = jax.lax.broadcasted_iota(jnp.int32, logits.shape, 1)
    v1 = jnp.max(logits, axis=-1, keepdims=True)
    i1 = jnp.min(jnp.where(logits == v1, iota, S), axis=-1, keepdims=True)
    l2 = jnp.where(iota == i1, -jnp.inf, logits)
    v2 = jnp.max(l2, axis=-1, keepdims=True)
    i2 = jnp.min(jnp.where(l2 == v2, iota, S), axis=-1, keepdims=True)
    e2 = jnp.exp(v2 - v1)
    w1 = 1.0 / (1.0 + e2)
    w2 = e2 * w1
    w12_ref[...] = jnp.concatenate([w1, w2], axis=1)

    oh0 = (iota == i1).astype(jnp.float32)
    oh1 = (iota == i2).astype(jnp.float32)
    ohsum = oh0 + oh1
    tri = (jax.lax.broadcasted_iota(jnp.int32, (CH, CH), 0)
           > jax.lax.broadcasted_iota(jnp.int32, (CH, CH), 1)
           ).astype(jnp.float32)
    carry = jnp.zeros((1, S), jnp.float32)
    bases = []
    for c in range(T // CH):
        chunk = ohsum[c * CH:(c + 1) * CH, :]
        excl = jnp.dot(tri, chunk, preferred_element_type=jnp.float32)
        bases.append(excl + carry)
        carry = carry + jnp.sum(chunk, axis=0, keepdims=True)
    base = jnp.concatenate(bases, axis=0)
    counts = carry
    nblk = jnp.floor((counts + (BR - 1)) * (1.0 / BR))
    end_blk = _lane_cumsum8(nblk)
    starts = (end_blk - nblk) * BR
    pos0 = jnp.sum(oh0 * (starts + base), axis=1, keepdims=True)
    pos1 = jnp.sum(oh1 * (starts + base + oh0), axis=1, keepdims=True)
    pos_ref[...] = jnp.concatenate([pos0, pos1], axis=1).astype(jnp.int32)
    j_col = jax.lax.broadcasted_iota(
        jnp.int32, (NBP, 1), 0).astype(jnp.float32)
    ind = (j_col >= end_blk).astype(jnp.float32)
    be = jnp.minimum(jnp.sum(ind, axis=1, keepdims=True), S - 1)
    be_ref[...] = be.astype(jnp.int32)


def _group_kernel(be_ref, normed_ref, posT_ref, w1_ref, b1_ref, w2_ref,
                  b2_ref, out_ref, xscr_ref):
    hb = pl.program_id(0)
    j = pl.program_id(1)
    row = j * BR

    @pl.when(hb == 0)
    def _dispatch():
        p0 = posT_ref[0]
        p1 = posT_ref[1]
        r_col = jax.lax.broadcasted_iota(jnp.int32, (BR, T), 0) + row
        onehot = jnp.where(
            jnp.logical_or(p0 == r_col, p1 == r_col), 1.0, 0.0).astype(_DOT)
        rows = jnp.dot(onehot, normed_ref[...],
                       preferred_element_type=jnp.float32)
        xscr_ref[pl.ds(row, BR), :] = rows.astype(_DOT)

    xb = xscr_ref[pl.ds(row, BR), :]
    h1 = jnp.dot(xb, w1_ref[0].astype(_DOT),
                 preferred_element_type=jnp.float32)
    h1 = h1 + b1_ref[0]
    g = _gelu(h1).astype(_DOT)
    y = jnp.dot(g, w2_ref[0].astype(_DOT), preferred_element_type=jnp.float32)

    @pl.when(hb == 0)
    def _init():
        out_ref[pl.ds(row, BR), :] = (y + b2_ref[0]).astype(out_ref.dtype)

    @pl.when(hb != 0)
    def _acc():
        out_ref[pl.ds(row, BR), :] += y.astype(out_ref.dtype)


def _dense_kernel(x_ref, ys_ref, pos_ref, w12_ref, g2_ref, b2_ref, wd1_ref,
                  bd1_ref, wd2_ref, bd2_ref, wg_ref, bg_ref, out_ref):
    x = x_ref[...]
    pos_b = pos_ref[...]
    w_b = w12_ref[...]
    p_row = jax.lax.broadcasted_iota(jnp.int32, (BT2, NP), 1)
    mask = (jnp.where(pos_b[:, 0:1] == p_row, w_b[:, 0:1], 0.0)
            + jnp.where(pos_b[:, 1:2] == p_row, w_b[:, 1:2], 0.0)).astype(_DOT)
    so = jnp.dot(mask, ys_ref[...], preferred_element_type=jnp.float32)
    x1 = x + so
    x1n = _layer_norm(x1, g2_ref[...], b2_ref[...])
    gate_logit = jnp.sum(x1n * wg_ref[...], axis=-1, keepdims=True) + bg_ref[0, 0]
    gate = jax.nn.sigmoid(gate_logit)
    h = jnp.dot(x1n.astype(_DOT), wd1_ref[...].astype(_DOT),
                preferred_element_type=jnp.float32) + bd1_ref[...]
    g = _gelu(h).astype(_DOT)
    do = jnp.dot(g, wd2_ref[...].astype(_DOT),
                 preferred_element_type=jnp.float32)
    do = do + bd2_ref[...]
    out_ref[0] = x1 + gate * so + (1.0 - gate) * do


def kernel(x, gamma1, beta1, gamma2, beta2, Wr, br, W1e, b1e, W2e, b2e,
           Wd1, bd1, Wd2, bd2, Wg, bg):
    x2d = x.reshape(T, D)

    normed, pos, w12, block_expert = pl.pallas_call(
        _router_kernel,
        out_shape=(
            jax.ShapeDtypeStruct((T, D), _DOT),
            jax.ShapeDtypeStruct((T, K), jnp.int32),
            jax.ShapeDtypeStruct((T, K), jnp.float32),
            jax.ShapeDtypeStruct((NBP, 1), jnp.int32),
        ),
    )(x2d, gamma1.reshape(1, D), beta1.reshape(1, D), Wr, br.reshape(1, S))

    posT = jnp.transpose(pos, (1, 0)).reshape(K, 1, T)

    y_sorted = pl.pallas_call(
        _group_kernel,
        grid_spec=pltpu.PrefetchScalarGridSpec(
            num_scalar_prefetch=1,
            grid=(NH, NB),
            in_specs=[
                pl.BlockSpec((T, D), lambda hb, j, be: (0, 0)),
                pl.BlockSpec((K, 1, T), lambda hb, j, be: (0, 0, 0)),
                pl.BlockSpec((1, D, BH), lambda hb, j, be: (be[j, 0], 0, hb)),
                pl.BlockSpec((1, 1, BH), lambda hb, j, be: (be[j, 0], 0, hb)),
                pl.BlockSpec((1, BH, D), lambda hb, j, be: (be[j, 0], hb, 0)),
                pl.BlockSpec((1, 1, D), lambda hb, j, be: (be[j, 0], 0, 0)),
            ],
            out_specs=pl.BlockSpec((NP, D), lambda hb, j, be: (0, 0)),
            scratch_shapes=[pltpu.VMEM((NP, D), _DOT)],
        ),
        out_shape=jax.ShapeDtypeStruct((NP, D), _DOT),
        compiler_params=pltpu.CompilerParams(
            dimension_semantics=("arbitrary", "arbitrary"),
        ),
    )(block_expert, normed, posT, W1e, b1e.reshape(S, 1, H), W2e,
      b2e.reshape(S, 1, D))

    out = pl.pallas_call(
        _dense_kernel,
        grid=(NT2,),
        in_specs=[
            pl.BlockSpec((BT2, D), lambda t: (t, 0)),
            pl.BlockSpec((NP, D), lambda t: (0, 0)),
            pl.BlockSpec((BT2, K), lambda t: (t, 0)),
            pl.BlockSpec((BT2, K), lambda t: (t, 0)),
            pl.BlockSpec((1, D), lambda t: (0, 0)),
            pl.BlockSpec((1, D), lambda t: (0, 0)),
            pl.BlockSpec((D, H), lambda t: (0, 0)),
            pl.BlockSpec((1, H), lambda t: (0, 0)),
            pl.BlockSpec((H, D), lambda t: (0, 0)),
            pl.BlockSpec((1, D), lambda t: (0, 0)),
            pl.BlockSpec((1, D), lambda t: (0, 0)),
            pl.BlockSpec((1, 1), lambda t: (0, 0)),
        ],
        out_specs=pl.BlockSpec((1, BT2, D), lambda t: (0, t, 0)),
        out_shape=jax.ShapeDtypeStruct((1, T, D), jnp.float32),
        compiler_params=pltpu.CompilerParams(
            dimension_semantics=("arbitrary",),
        ),
    )(
        x2d,
        y_sorted,
        pos,
        w12,
        gamma2.reshape(1, D),
        beta2.reshape(1, D),
        Wd1,
        bd1.reshape(1, H),
        Wd2,
        bd2.reshape(1, D),
        Wg.reshape(1, D),
        bg.reshape(1, 1),
    )
    return out


# speedup vs baseline: 1.2061x; 1.0454x over previous
import jax
import jax.numpy as jnp
from jax.experimental import pallas as pl
from jax.experimental.pallas import tpu as pltpu

T = 2048
D = 1024
H = 4096
S = 8
K = 2
A = T * K
BR = 256
NB = A // BR + S - 1
NP = NB * BR
NBP = 32
CH = 512
EPS = 1e-5

BH = 2048
NH = H // BH
BT2 = 256
NT2 = T // BT2

_DOT = jnp.bfloat16


def _layer_norm(x, g, b):
    mu = jnp.mean(x, axis=-1, keepdims=True)
    var = jnp.mean((x - mu) ** 2, axis=-1, keepdims=True)
    return (x - mu) * jax.lax.rsqrt(var + EPS) * g + b


def _gelu(x):
    return 0.5 * x * (1.0 + jax.lax.erf(x * 0.7071067811865476))


def _lane_cumsum8(v):
    for sh in (1, 2, 4):
        v = v + jnp.concatenate(
            [jnp.zeros((1, sh), v.dtype), v[:, : S - sh]], axis=1)
    return v


def _router_kernel(x_ref, g1_ref, b1_ref, wr_ref, br_ref,
                   normed_ref, pos_ref, w12_ref, be_ref):
    x = x_ref[...]
    normed = _layer_norm(x, g1_ref[...], b1_ref[...])
    normed_ref[...] = normed.astype(normed_ref.dtype)
    logits = jnp.dot(normed, wr_ref[...], preferred_element_type=jnp.float32)
    logits = logits + br_ref[...]
    iota = jax.lax.broadcasted_iota(jnp.int32, logits.shape, 1)
    v1 = jnp.max(logits, axis=-1, keepdims=True)
    i1 = jnp.min(jnp.where(logits == v1, iota, S), axis=-1, keepdims=True)
    l2 = jnp.where(iota == i1, -jnp.inf, logits)
    v2 = jnp.max(l2, axis=-1, keepdims=True)
    i2 = jnp.min(jnp.where(l2 == v2, iota, S), axis=-1, keepdims=True)
    e2 = jnp.exp(v2 - v1)
    w1 = 1.0 / (1.0 + e2)
    w2 = e2 * w1
    w12_ref[...] = jnp.concatenate([w1, w2], axis=1)

    oh0 = (iota == i1).astype(jnp.float32)
    oh1 = (iota == i2).astype(jnp.float32)
    ohsum = oh0 + oh1
    tri = (jax.lax.broadcasted_iota(jnp.int32, (CH, CH), 0)
           > jax.lax.broadcasted_iota(jnp.int32, (CH, CH), 1)
           ).astype(jnp.float32)
    carry = jnp.zeros((1, S), jnp.float32)
    bases = []
    for c in range(T // CH):
        chunk = ohsum[c * CH:(c + 1) * CH, :]
        excl = jnp.dot(tri, chunk, preferred_element_type=jnp.float32)
        bases.append(excl + carry)
        carry = carry + jnp.sum(chunk, axis=0, keepdims=True)
    base = jnp.concatenate(bases, axis=0)
    counts = carry
    nblk = jnp.floor((counts + (BR - 1)) * (1.0 / BR))
    end_blk = _lane_cumsum8(nblk)
    starts = (end_blk - nblk) * BR
    pos0 = jnp.sum(oh0 * (starts + base), axis=1, keepdims=True)
    pos1 = jnp.sum(oh1 * (starts + base + oh0), axis=1, keepdims=True)
    pos_ref[...] = jnp.concatenate([pos0, pos1], axis=1).astype(jnp.int32)
    j_col = jax.lax.broadcasted_iota(
        jnp.int32, (NBP, 1), 0).astype(jnp.float32)
    ind = (j_col >= end_blk).astype(jnp.float32)
    be = jnp.minimum(jnp.sum(ind, axis=1, keepdims=True), S - 1)
    be_ref[...] = be.astype(jnp.int32)


def _group_kernel(be_ref, normed_ref, posT_ref, w1_ref, b1_ref, w2_ref,
                  b2_ref, out_ref):
    hb = pl.program_id(0)
    j = pl.program_id(1)
    row = j * BR

    p0 = posT_ref[0]
    p1 = posT_ref[1]
    r_col = jax.lax.broadcasted_iota(jnp.int32, (BR, T), 0) + row
    onehot = jnp.where(
        jnp.logical_or(p0 == r_col, p1 == r_col), 1.0, 0.0).astype(_DOT)
    xb = jnp.dot(onehot, normed_ref[...],
                 preferred_element_type=jnp.float32).astype(_DOT)
    h1 = jnp.dot(xb, w1_ref[0].astype(_DOT),
                 preferred_element_type=jnp.float32)
    h1 = h1 + b1_ref[0]
    g = _gelu(h1).astype(_DOT)
    y = jnp.dot(g, w2_ref[0].astype(_DOT), preferred_element_type=jnp.float32)

    @pl.when(hb == 0)
    def _init():
        out_ref[pl.ds(row, BR), :] = (y + b2_ref[0]).astype(out_ref.dtype)

    @pl.when(hb != 0)
    def _acc():
        out_ref[pl.ds(row, BR), :] += y.astype(out_ref.dtype)


def _dense_kernel(x_ref, ys_ref, pos_ref, w12_ref, g2_ref, b2_ref, wd1_ref,
                  bd1_ref, wd2_ref, bd2_ref, wg_ref, bg_ref, out_ref):
    x = x_ref[...]
    pos_b = pos_ref[...]
    w_b = w12_ref[...]
    p_row = jax.lax.broadcasted_iota(jnp.int32, (BT2, NP), 1)
    mask = (jnp.where(pos_b[:, 0:1] == p_row, w_b[:, 0:1], 0.0)
            + jnp.where(pos_b[:, 1:2] == p_row, w_b[:, 1:2], 0.0)).astype(_DOT)
    so = jnp.dot(mask, ys_ref[...], preferred_element_type=jnp.float32)
    x1 = x + so
    x1n = _layer_norm(x1, g2_ref[...], b2_ref[...])
    gate_logit = jnp.sum(x1n * wg_ref[...], axis=-1, keepdims=True) + bg_ref[0, 0]
    gate = jax.nn.sigmoid(gate_logit)
    h = jnp.dot(x1n.astype(_DOT), wd1_ref[...].astype(_DOT),
                preferred_element_type=jnp.float32) + bd1_ref[...]
    g = _gelu(h).astype(_DOT)
    do = jnp.dot(g, wd2_ref[...].astype(_DOT),
                 preferred_element_type=jnp.float32)
    do = do + bd2_ref[...]
    out_ref[0] = x1 + gate * so + (1.0 - gate) * do


def kernel(x, gamma1, beta1, gamma2, beta2, Wr, br, W1e, b1e, W2e, b2e,
           Wd1, bd1, Wd2, bd2, Wg, bg):
    x2d = x.reshape(T, D)

    normed, pos, w12, block_expert = pl.pallas_call(
        _router_kernel,
        out_shape=(
            jax.ShapeDtypeStruct((T, D), _DOT),
            jax.ShapeDtypeStruct((T, K), jnp.int32),
            jax.ShapeDtypeStruct((T, K), jnp.float32),
            jax.ShapeDtypeStruct((NBP, 1), jnp.int32),
        ),
    )(x2d, gamma1.reshape(1, D), beta1.reshape(1, D), Wr, br.reshape(1, S))

    posT = jnp.transpose(pos, (1, 0)).reshape(K, 1, T)

    y_sorted = pl.pallas_call(
        _group_kernel,
        grid_spec=pltpu.PrefetchScalarGridSpec(
            num_scalar_prefetch=1,
            grid=(NH, NB),
            in_specs=[
                pl.BlockSpec((T, D), lambda hb, j, be: (0, 0)),
                pl.BlockSpec((K, 1, T), lambda hb, j, be: (0, 0, 0)),
                pl.BlockSpec((1, D, BH), lambda hb, j, be: (be[j, 0], 0, hb)),
                pl.BlockSpec((1, 1, BH), lambda hb, j, be: (be[j, 0], 0, hb)),
                pl.BlockSpec((1, BH, D), lambda hb, j, be: (be[j, 0], hb, 0)),
                pl.BlockSpec((1, 1, D), lambda hb, j, be: (be[j, 0], 0, 0)),
            ],
            out_specs=pl.BlockSpec((NP, D), lambda hb, j, be: (0, 0)),
        ),
        out_shape=jax.ShapeDtypeStruct((NP, D), _DOT),
        compiler_params=pltpu.CompilerParams(
            dimension_semantics=("arbitrary", "arbitrary"),
        ),
    )(block_expert, normed, posT, W1e, b1e.reshape(S, 1, H), W2e,
      b2e.reshape(S, 1, D))

    out = pl.pallas_call(
        _dense_kernel,
        grid=(NT2,),
        in_specs=[
            pl.BlockSpec((BT2, D), lambda t: (t, 0)),
            pl.BlockSpec((NP, D), lambda t: (0, 0)),
            pl.BlockSpec((BT2, K), lambda t: (t, 0)),
            pl.BlockSpec((BT2, K), lambda t: (t, 0)),
            pl.BlockSpec((1, D), lambda t: (0, 0)),
            pl.BlockSpec((1, D), lambda t: (0, 0)),
            pl.BlockSpec((D, H), lambda t: (0, 0)),
            pl.BlockSpec((1, H), lambda t: (0, 0)),
            pl.BlockSpec((H, D), lambda t: (0, 0)),
            pl.BlockSpec((1, D), lambda t: (0, 0)),
            pl.BlockSpec((1, D), lambda t: (0, 0)),
            pl.BlockSpec((1, 1), lambda t: (0, 0)),
        ],
        out_specs=pl.BlockSpec((1, BT2, D), lambda t: (0, t, 0)),
        out_shape=jax.ShapeDtypeStruct((1, T, D), jnp.float32),
        compiler_params=pltpu.CompilerParams(
            dimension_semantics=("arbitrary",),
        ),
    )(
        x2d,
        y_sorted,
        pos,
        w12,
        gamma2.reshape(1, D),
        beta2.reshape(1, D),
        Wd1,
        bd1.reshape(1, H),
        Wd2,
        bd2.reshape(1, D),
        Wg.reshape(1, D),
        bg.reshape(1, 1),
    )
    return out
